# baseline (speedup 1.0000x reference)
import jax
import jax.numpy as jnp
from jax.experimental import pallas as pl

_ACCELERATION = 4.0
_CENTER_FRACTION = 0.08
_W = 320
_SLOPE = 10.0
_NUM_ACTIONS = _W
_BUDGET = int(_NUM_ACTIONS / _ACCELERATION - _NUM_ACTIONS * _CENTER_FRACTION)

_ROWS_PER_BLOCK = 2048


def _policy_kernel(mask2d_ref, sampler_ref, u_ref, mpm_ref, rowval_ref):
    m = mask2d_ref[...]
    s = sampler_ref[...]
    u = u_ref[...]
    b = m.shape[0]
    prob = jnp.logaddexp(0.0, _SLOPE * s) / _SLOPE
    prob = jnp.broadcast_to(prob, (b, _W))
    denom = jnp.max((1.0 - m) * prob, axis=1, keepdims=True)
    prob = prob / denom
    mpm = prob * (1.0 - m)
    sel = m == 0
    x = jnp.where(sel, mpm, 0.0)
    sparsity = _BUDGET / _W
    xbar = jnp.mean(x, axis=1, keepdims=True)
    r = sparsity / xbar
    beta = (1.0 - sparsity) / (1.0 - xbar)
    le = (r <= 1.0).astype(x.dtype)
    normed = le * x * r + (1.0 - le) * (1.0 - (1.0 - x) * beta)
    mpm = jnp.where(sel, normed, mpm)
    hard = (mpm > u).astype(mpm.dtype)
    mpm_ref[...] = mpm
    rowval_ref[...] = m + hard


def _mask_kernel(ks_ref, rv_ref, out_ref):
    out_ref[...] = ks_ref[...] * rv_ref[...]


def kernel(mask, kspace, sampler):
    batch, coils, slc, height, width, _ = kspace.shape
    mask2d = mask[:, :, 0, 0, :, :].reshape(batch, width)
    u = jax.random.uniform(jax.random.key(1), (batch, width), dtype=kspace.dtype)

    mpm, rowval = pl.pallas_call(
        _policy_kernel,
        out_shape=[
            jax.ShapeDtypeStruct((batch, width), kspace.dtype),
            jax.ShapeDtypeStruct((batch, width), kspace.dtype),
        ],
    )(mask2d, sampler, u)

    ksv = jnp.swapaxes(kspace, 4, 5)
    rv6 = rowval.reshape(batch, 1, 1, 1, 1, width)
    _S = 4
    grid = (batch, coils, slc // _S)
    out6 = pl.pallas_call(
        _mask_kernel,
        grid=grid,
        in_specs=[
            pl.BlockSpec((1, 1, _S, height, 2, width), lambda b, c, s: (b, c, s, 0, 0, 0)),
            pl.BlockSpec((1, 1, 1, 1, 1, width), lambda b, c, s: (b, 0, 0, 0, 0, 0)),
        ],
        out_specs=pl.BlockSpec((1, 1, _S, height, 2, width), lambda b, c, s: (b, c, s, 0, 0, 0)),
        out_shape=jax.ShapeDtypeStruct((batch, coils, slc, height, 2, width), kspace.dtype),
    )(ksv, rv6)
    masked_kspace = jnp.swapaxes(out6, 4, 5)

    mask_new = jnp.broadcast_to(
        rowval.reshape(batch, 1, 1, 1, width, 1), (batch, 1, 1, height, width, 1)
    )
    final_prob_mask = jnp.broadcast_to(
        mpm.reshape(batch, 1, 1, 1, width, 1), (batch, 1, 1, height, width, 1)
    )
    return (masked_kspace, mask, mask_new, final_prob_mask)


# speedup vs baseline: 1.0710x; 1.0710x over previous
import jax
import jax.numpy as jnp
from jax.experimental import pallas as pl

_ACCELERATION = 4.0
_CENTER_FRACTION = 0.08
_W = 320
_SLOPE = 10.0
_NUM_ACTIONS = _W
_BUDGET = int(_NUM_ACTIONS / _ACCELERATION - _NUM_ACTIONS * _CENTER_FRACTION)


def _policy_kernel(mask2d_ref, sampler_ref, u_ref, mpm_ref, rowval_ref):
    m = mask2d_ref[...]
    s = sampler_ref[...]
    u = u_ref[...]
    b = m.shape[0]
    prob = jnp.logaddexp(0.0, _SLOPE * s) / _SLOPE
    prob = jnp.broadcast_to(prob, (b, _W))
    denom = jnp.max((1.0 - m) * prob, axis=1, keepdims=True)
    prob = prob / denom
    mpm = prob * (1.0 - m)
    sel = m == 0
    x = jnp.where(sel, mpm, 0.0)
    sparsity = _BUDGET / _W
    xbar = jnp.mean(x, axis=1, keepdims=True)
    r = sparsity / xbar
    beta = (1.0 - sparsity) / (1.0 - xbar)
    le = (r <= 1.0).astype(x.dtype)
    normed = le * x * r + (1.0 - le) * (1.0 - (1.0 - x) * beta)
    mpm = jnp.where(sel, normed, mpm)
    hard = (mpm > u).astype(mpm.dtype)
    mpm_ref[...] = mpm
    rowval_ref[...] = m + hard


def _mask_kernel(ks_ref, rv_ref, mpm_ref, mask_ref,
                 out_ref, mask_out_ref, mask_new_ref, fpm_ref):
    c = pl.program_id(1)
    out_ref[...] = ks_ref[...] * rv_ref[...]

    @pl.when(c == 0)
    def _():
        mask_out_ref[...] = mask_ref[...]
        mask_new_ref[...] = jnp.broadcast_to(rv_ref[...], mask_new_ref.shape)
        fpm_ref[...] = jnp.broadcast_to(mpm_ref[...], fpm_ref.shape)


def kernel(mask, kspace, sampler):
    batch, coils, slc, height, width, _ = kspace.shape
    mask2d = mask[:, :, 0, 0, :, :].reshape(batch, width)
    u = jax.random.uniform(jax.random.key(1), (batch, width), dtype=kspace.dtype)

    mpm, rowval = pl.pallas_call(
        _policy_kernel,
        out_shape=[
            jax.ShapeDtypeStruct((batch, width), kspace.dtype),
            jax.ShapeDtypeStruct((batch, width), kspace.dtype),
        ],
    )(mask2d, sampler, u)

    ksv = jnp.swapaxes(kspace, 4, 5)
    maskv = jnp.swapaxes(mask, 4, 5)
    rv6 = rowval.reshape(batch, 1, 1, 1, 1, width)
    mpm6 = mpm.reshape(batch, 1, 1, 1, 1, width)
    grid = (batch, coils)
    row_spec = pl.BlockSpec((1, 1, 1, 1, 1, width), lambda b, c: (b, 0, 0, 0, 0, 0))
    big_spec = pl.BlockSpec((1, 1, slc, height, 2, width), lambda b, c: (b, c, 0, 0, 0, 0))
    mask_spec = pl.BlockSpec((1, 1, slc, height, 1, width), lambda b, c: (b, 0, 0, 0, 0, 0))
    bcast_spec = pl.BlockSpec((1, 1, 1, height, 1, width), lambda b, c: (b, 0, 0, 0, 0, 0))
    out6, mask_o, mask_new6, fpm6 = pl.pallas_call(
        _mask_kernel,
        grid=grid,
        in_specs=[big_spec, row_spec, row_spec, mask_spec],
        out_specs=[big_spec, mask_spec, bcast_spec, bcast_spec],
        out_shape=[
            jax.ShapeDtypeStruct((batch, coils, slc, height, 2, width), kspace.dtype),
            jax.ShapeDtypeStruct((batch, 1, slc, height, 1, width), mask.dtype),
            jax.ShapeDtypeStruct((batch, 1, 1, height, 1, width), kspace.dtype),
            jax.ShapeDtypeStruct((batch, 1, 1, height, 1, width), kspace.dtype),
        ],
    )(ksv, rv6, mpm6, maskv)
    masked_kspace = jnp.swapaxes(out6, 4, 5)
    mask_in = jnp.swapaxes(mask_o, 4, 5)
    mask_new = jnp.swapaxes(mask_new6, 4, 5)
    final_prob_mask = jnp.swapaxes(fpm6, 4, 5)
    return (masked_kspace, mask_in, mask_new, final_prob_mask)
